# Initial kernel scaffold; baseline (speedup 1.0000x reference)
import jax
import jax.numpy as jnp
from jax.experimental import pallas as pl


def kernel(users, products, neighbors, weight, bias):
    raise NotImplementedError("write your pallas kernel here")


# speedup vs baseline: 4.6681x; 4.6681x over previous
import functools

import jax
import jax.numpy as jnp
from jax import lax
from jax.experimental import pallas as pl
from jax.experimental.pallas import tpu as pltpu
from jax.experimental.pallas import tpu_sc as plsc

N_NODES = 5000
D = 128
E = 320000
SLOPE = 0.2

NPAD = 5120
ROWS_PER_TILE = NPAD // 16
CH = 80
CHUNKS_PER_TILE = E // (16 * CH)
WB = ROWS_PER_TILE // CH

_mesh = plsc.VectorSubcoreMesh(core_axis_name="c", subcore_axis_name="s")


def _make_agg_body(with_deg):
    def _agg_body(tab, nbp, zrow, one_hbm, *refs):
        if with_deg:
            (agg_out, deg_out, idx2, rows, ones_v, agg_sh, deg_sh, sem) = refs
        else:
            (agg_out, idx2, rows, agg_sh, sem) = refs
        cid = lax.axis_index("c")
        sid = lax.axis_index("s")
        r0 = sid * ROWS_PER_TILE

        pltpu.sync_copy(zrow, rows)
        for k in range(WB):
            pltpu.sync_copy(rows, agg_sh.at[pl.ds(r0 + k * CH, CH)])
            if with_deg:
                pltpu.sync_copy(rows, deg_sh.at[pl.ds(r0 + k * CH, CH)])
        if with_deg:
            pltpu.sync_copy(one_hbm, ones_v)
        plsc.subcore_barrier()

        gt = 1 - cid
        base = sid * CHUNKS_PER_TILE

        def step(j, carry):
            pltpu.sync_copy(nbp.at[base + j], idx2)
            pltpu.async_copy(tab.at[gt].at[idx2.at[gt]], rows, sem).wait()
            pltpu.sync_copy(rows, agg_sh.at[idx2.at[cid]], add=True)
            if with_deg:
                pltpu.sync_copy(ones_v, deg_sh.at[idx2.at[cid]], add=True)
            return carry

        lax.fori_loop(0, CHUNKS_PER_TILE, step, 0)
        plsc.subcore_barrier()

        for k in range(WB):
            pltpu.sync_copy(agg_sh.at[pl.ds(r0 + k * CH, CH)], rows)
            pltpu.sync_copy(rows, agg_out.at[cid].at[pl.ds(r0 + k * CH, CH)])
            if with_deg:
                pltpu.sync_copy(deg_sh.at[pl.ds(r0 + k * CH, CH)], rows)
                pltpu.sync_copy(rows, deg_out.at[cid].at[pl.ds(r0 + k * CH, CH)])

    return _agg_body


_agg_deg_call = functools.partial(
    pl.kernel,
    out_type=[
        jax.ShapeDtypeStruct((2, NPAD, D), jnp.float32),
        jax.ShapeDtypeStruct((2, NPAD, D), jnp.float32),
    ],
    mesh=_mesh,
    scratch_types=[
        pltpu.VMEM((2, CH), jnp.int32),
        pltpu.VMEM((CH, D), jnp.float32),
        pltpu.VMEM((CH, D), jnp.float32),
        pltpu.VMEM_SHARED((NPAD, D), jnp.float32),
        pltpu.VMEM_SHARED((NPAD, D), jnp.float32),
        pltpu.SemaphoreType.DMA,
    ],
)(_make_agg_body(True))

_agg_call = functools.partial(
    pl.kernel,
    out_type=jax.ShapeDtypeStruct((2, NPAD, D), jnp.float32),
    mesh=_mesh,
    scratch_types=[
        pltpu.VMEM((2, CH), jnp.int32),
        pltpu.VMEM((CH, D), jnp.float32),
        pltpu.VMEM_SHARED((NPAD, D), jnp.float32),
        pltpu.SemaphoreType.DMA,
    ],
)(_make_agg_body(False))


def _dense_body(x_ref, agg_ref, deg_ref, w_ref, b_ref, o_ref):
    x = x_ref[0]
    agg = agg_ref[0]
    deg = jnp.maximum(deg_ref[0, :, 0:1], 1.0)
    h = x + agg / deg
    y = jnp.dot(h, w_ref[...], preferred_element_type=jnp.float32,
                precision=lax.Precision.HIGHEST) + b_ref[...]
    o_ref[0] = jnp.where(y >= 0, y, SLOPE * y)


def _dense(x, agg, deg, w, b2):
    rb = 1000
    grid = (2, N_NODES // rb)
    return pl.pallas_call(
        _dense_body,
        grid=grid,
        in_specs=[
            pl.BlockSpec((1, rb, D), lambda i, j: (i, j, 0)),
            pl.BlockSpec((1, rb, D), lambda i, j: (i, j, 0)),
            pl.BlockSpec((1, rb, D), lambda i, j: (i, j, 0)),
            pl.BlockSpec((D, D), lambda i, j: (0, 0)),
            pl.BlockSpec((1, D), lambda i, j: (0, 0)),
        ],
        out_specs=pl.BlockSpec((1, rb, D), lambda i, j: (i, j, 0)),
        out_shape=jax.ShapeDtypeStruct((2, N_NODES, D), jnp.float32),
    )(x, agg, deg, w, b2)


def kernel(users, products, neighbors, weight, bias):
    nbp = neighbors.astype(jnp.int32).reshape(2, E // CH, CH).transpose(1, 0, 2)
    x = jnp.stack([users, products])
    b2 = bias.reshape(1, D)
    zrow = jnp.zeros((CH, D), jnp.float32)
    ones = jnp.ones((CH, D), jnp.float32)

    agg1, deg = _agg_deg_call(x, nbp, zrow, ones)
    deg_s = deg[:, :N_NODES]
    x = _dense(x, agg1[:, :N_NODES], deg_s, weight, b2)
    agg2 = _agg_call(x, nbp, zrow, ones)
    x = _dense(x, agg2[:, :N_NODES], deg_s, weight, b2)
    return x[0], x[1]


# speedup vs baseline: 7.8356x; 1.6785x over previous
import functools

import jax
import jax.numpy as jnp
from jax import lax
from jax.experimental import pallas as pl
from jax.experimental.pallas import tpu as pltpu
from jax.experimental.pallas import tpu_sc as plsc

N_NODES = 5000
D = 128
E = 320000
SLOPE = 0.2

NPAD = 5120
ROWS_PER_TILE = NPAD // 16
CH = 80
CHUNKS_PER_TILE = E // (16 * CH)
WB = ROWS_PER_TILE // CH

_mesh = plsc.VectorSubcoreMesh(core_axis_name="c", subcore_axis_name="s")


def _make_agg_body(with_deg):
    def _agg_body(tab, nbp, zrow, one_hbm, *refs):
        if with_deg:
            (agg_out, deg_out, idx0, idx1, rows0, rows1, ones_v,
             agg_sh, deg_sh, sem0, sem1) = refs
        else:
            (agg_out, idx0, idx1, rows0, rows1, agg_sh, sem0, sem1) = refs
        cid = lax.axis_index("c")
        sid = lax.axis_index("s")
        r0 = sid * ROWS_PER_TILE

        pltpu.sync_copy(zrow, rows0)
        for k in range(WB):
            pltpu.sync_copy(rows0, agg_sh.at[pl.ds(r0 + k * CH, CH)])
            if with_deg:
                pltpu.sync_copy(rows0, deg_sh.at[pl.ds(r0 + k * CH, CH)])
        if with_deg:
            pltpu.sync_copy(one_hbm, ones_v)
        plsc.subcore_barrier()

        gt = 1 - cid
        base = sid * CHUNKS_PER_TILE
        half = CHUNKS_PER_TILE // 2

        def scatter(rows, idx):
            pltpu.sync_copy(rows, agg_sh.at[idx.at[cid]], add=True)
            if with_deg:
                pltpu.sync_copy(ones_v, deg_sh.at[idx.at[cid]], add=True)

        pltpu.sync_copy(nbp.at[base], idx0)
        pltpu.async_copy(tab.at[gt].at[idx0.at[gt]], rows0, sem0)

        def step(i, carry):
            j = base + 2 * i
            pltpu.sync_copy(nbp.at[j + 1], idx1)
            pltpu.async_copy(tab.at[gt].at[idx1.at[gt]], rows1, sem1)
            pltpu.make_async_copy(tab.at[gt].at[idx0.at[gt]], rows0, sem0).wait()
            scatter(rows0, idx0)

            @pl.when(i < half - 1)
            def _():
                pltpu.sync_copy(nbp.at[j + 2], idx0)
                pltpu.async_copy(tab.at[gt].at[idx0.at[gt]], rows0, sem0)

            pltpu.make_async_copy(tab.at[gt].at[idx1.at[gt]], rows1, sem1).wait()
            scatter(rows1, idx1)
            return carry

        lax.fori_loop(0, half, step, 0)
        plsc.subcore_barrier()

        for k in range(WB):
            pltpu.sync_copy(agg_sh.at[pl.ds(r0 + k * CH, CH)], rows0)
            pltpu.sync_copy(rows0, agg_out.at[cid].at[pl.ds(r0 + k * CH, CH)])
            if with_deg:
                pltpu.sync_copy(deg_sh.at[pl.ds(r0 + k * CH, CH)], rows0)
                pltpu.sync_copy(rows0, deg_out.at[cid].at[pl.ds(r0 + k * CH, CH)])

    return _agg_body


_agg_deg_call = functools.partial(
    pl.kernel,
    out_type=[
        jax.ShapeDtypeStruct((2, NPAD, D), jnp.float32),
        jax.ShapeDtypeStruct((2, NPAD, D), jnp.float32),
    ],
    mesh=_mesh,
    scratch_types=[
        pltpu.VMEM((2, CH), jnp.int32),
        pltpu.VMEM((2, CH), jnp.int32),
        pltpu.VMEM((CH, D), jnp.float32),
        pltpu.VMEM((CH, D), jnp.float32),
        pltpu.VMEM((CH, D), jnp.float32),
        pltpu.VMEM_SHARED((NPAD, D), jnp.float32),
        pltpu.VMEM_SHARED((NPAD, D), jnp.float32),
        pltpu.SemaphoreType.DMA,
        pltpu.SemaphoreType.DMA,
    ],
)(_make_agg_body(True))

_agg_call = functools.partial(
    pl.kernel,
    out_type=jax.ShapeDtypeStruct((2, NPAD, D), jnp.float32),
    mesh=_mesh,
    scratch_types=[
        pltpu.VMEM((2, CH), jnp.int32),
        pltpu.VMEM((2, CH), jnp.int32),
        pltpu.VMEM((CH, D), jnp.float32),
        pltpu.VMEM((CH, D), jnp.float32),
        pltpu.VMEM_SHARED((NPAD, D), jnp.float32),
        pltpu.SemaphoreType.DMA,
        pltpu.SemaphoreType.DMA,
    ],
)(_make_agg_body(False))


def _dense_body(x_ref, agg_ref, deg_ref, w_ref, b_ref, o_ref):
    x = x_ref[0]
    agg = agg_ref[0]
    deg = jnp.maximum(deg_ref[0, :, 0:1], 1.0)
    h = x + agg / deg
    y = jnp.dot(h, w_ref[...], preferred_element_type=jnp.float32,
                precision=lax.Precision.HIGHEST) + b_ref[...]
    o_ref[0] = jnp.where(y >= 0, y, SLOPE * y)


def _dense(x, agg, deg, w, b2):
    rb = 1000
    grid = (2, N_NODES // rb)
    return pl.pallas_call(
        _dense_body,
        grid=grid,
        in_specs=[
            pl.BlockSpec((1, rb, D), lambda i, j: (i, j, 0)),
            pl.BlockSpec((1, rb, D), lambda i, j: (i, j, 0)),
            pl.BlockSpec((1, rb, D), lambda i, j: (i, j, 0)),
            pl.BlockSpec((D, D), lambda i, j: (0, 0)),
            pl.BlockSpec((1, D), lambda i, j: (0, 0)),
        ],
        out_specs=pl.BlockSpec((1, rb, D), lambda i, j: (i, j, 0)),
        out_shape=jax.ShapeDtypeStruct((2, N_NODES, D), jnp.float32),
    )(x, agg, deg, w, b2)


def kernel(users, products, neighbors, weight, bias):
    nbp = neighbors.astype(jnp.int32).reshape(2, E // CH, CH).transpose(1, 0, 2)
    x = jnp.stack([users, products])
    b2 = bias.reshape(1, D)
    zrow = jnp.zeros((CH, D), jnp.float32)
    ones = jnp.ones((CH, D), jnp.float32)

    agg1, deg = _agg_deg_call(x, nbp, zrow, ones)
    deg_s = deg[:, :N_NODES]
    x = _dense(x, agg1[:, :N_NODES], deg_s, weight, b2)
    agg2 = _agg_call(x, nbp, zrow, ones)
    x = _dense(x, agg2[:, :N_NODES], deg_s, weight, b2)
    return x[0], x[1]
